# Initial kernel scaffold; baseline (speedup 1.0000x reference)
import jax
import jax.numpy as jnp
from jax.experimental import pallas as pl


def kernel(pc, im, params):
    raise NotImplementedError("write your pallas kernel here")


# speedup vs baseline: 1.0013x; 1.0013x over previous
import functools

import jax
import jax.numpy as jnp
from jax.experimental import pallas as pl
from jax.experimental.pallas import tpu as pltpu

D = 192
NHEADS = 12
DEPTH = 3
PATCH = 14
SCALE = 2
GROUP_SIZE = 32
RADIUS = 0.1


def _ln(x, g, b, eps):
    m = x.mean(-1, keepdims=True)
    v = x.var(-1, keepdims=True)
    return (x - m) / jnp.sqrt(v + eps) * g + b


def _inorm(x):
    m = x.mean(axis=(1, 2), keepdims=True)
    v = x.var(axis=(1, 2), keepdims=True)
    return (x - m) / jnp.sqrt(v + 1e-5)


def _gelu(x):
    return jax.nn.gelu(x, approximate=False)


def _fps(p, n_samples):
    p = jax.lax.stop_gradient(p)
    B, N, _ = p.shape
    start = jnp.zeros((B,), jnp.int32)
    dists0 = jnp.full((B, N), 1e10, dtype=p.dtype)
    idx_buf = jnp.zeros((B, n_samples), jnp.int32).at[:, 0].set(start)

    def body(i, state):
        dists, buf, last = state
        last_p = jnp.take_along_axis(p, last[:, None, None], axis=1)
        d = jnp.sum((p - last_p) ** 2, axis=-1)
        dists = jnp.minimum(dists, d)
        nxt = jnp.argmax(dists, axis=-1).astype(jnp.int32)
        buf = buf.at[:, i].set(nxt)
        return (dists, buf, nxt)

    _, buf, _ = jax.lax.fori_loop(1, n_samples, body, (dists0, idx_buf, start))
    return buf


def _ball_query(support, centers, radius, nsample):
    support = jax.lax.stop_gradient(support)
    centers = jax.lax.stop_gradient(centers)
    B, N, _ = support.shape
    d = (jnp.sum(centers ** 2, -1)[:, :, None]
         + jnp.sum(support ** 2, -1)[:, None, :]
         - 2.0 * jnp.einsum('bmd,bnd->bmn', centers, support))
    idx = jnp.broadcast_to(jnp.arange(N, dtype=jnp.int32), d.shape)
    idx = jnp.where(d > radius * radius, N, idx)
    idx = jnp.sort(idx, axis=-1)[:, :, :nsample]
    first = idx[:, :, :1]
    idx = jnp.where(idx == N, jnp.broadcast_to(first, idx.shape), idx)
    return idx


def _gather(x, idx):
    return jnp.take_along_axis(x, idx[..., None], axis=1)


def _group(x, idx):
    B, M, K = idx.shape
    g = jnp.take_along_axis(x, idx.reshape(B, M * K)[..., None], axis=1)
    return g.reshape(B, M, K, x.shape[-1])


def _conv_block(x, w):
    return jax.nn.relu(_inorm(x @ w))


def _channel_attn(x, w1, w2):
    s = x.mean(axis=(1, 2))
    a = jax.nn.sigmoid(jax.nn.relu(s @ w1) @ w2)
    return x * a[:, None, None, :]


def _pc_stage(cur_p, cur_f, sp):
    B, N, _ = cur_p.shape
    M = N // SCALE
    idx = _fps(cur_p, M)
    center_p = _gather(cur_p, idx)
    center_f = _gather(cur_f, idx)
    nidx = _ball_query(cur_p, center_p, RADIUS, GROUP_SIZE)
    dp = _group(cur_p, nidx) - center_p[:, :, None, :]
    fj = _group(cur_f, nidx)
    df = fj - center_f[:, :, None, :]
    h = jnp.concatenate([dp, df], axis=-1)
    h = _channel_attn(_conv_block(h, sp['w1']), sp['ca1_w1'], sp['ca1_w2'])
    pooled = jnp.broadcast_to(h.max(axis=2, keepdims=True), h.shape)
    h = jnp.concatenate([pooled, h], axis=-1)
    h = _channel_attn(_conv_block(h, sp['w2']), sp['ca2_w1'], sp['ca2_w2'])
    return center_p, h.max(axis=2)


def _vit_block(x, bp):
    B, T, C = x.shape
    hd = C // NHEADS
    h = _ln(x, bp['ln1_g'], bp['ln1_b'], 1e-5)
    qkv = h @ bp['qkv_w'] + bp['qkv_b']
    q, k, v = jnp.split(qkv, 3, axis=-1)
    q = q.reshape(B, T, NHEADS, hd).transpose(0, 2, 1, 3)
    k = k.reshape(B, T, NHEADS, hd).transpose(0, 2, 1, 3)
    v = v.reshape(B, T, NHEADS, hd).transpose(0, 2, 1, 3)
    attn = jax.nn.softmax((q @ k.transpose(0, 1, 3, 2)) * (hd ** -0.5), axis=-1)
    o = (attn @ v).transpose(0, 2, 1, 3).reshape(B, T, C)
    x = x + (o @ bp['proj_w'] + bp['proj_b'])
    h = _ln(x, bp['ln2_g'], bp['ln2_b'], 1e-5)
    x = x + (_gelu(h @ bp['fc1_w'] + bp['fc1_b']) @ bp['fc2_w'] + bp['fc2_b'])
    return x


def _cross_mha(q_in, kv_in, cp):
    B, Tq, C = q_in.shape
    Tk = kv_in.shape[1]
    hd = C // NHEADS
    q = q_in @ cp['in_w'][:, :C] + cp['in_b'][:C]
    k = kv_in @ cp['in_w'][:, C:2 * C] + cp['in_b'][C:2 * C]
    v = kv_in @ cp['in_w'][:, 2 * C:] + cp['in_b'][2 * C:]
    q = q.reshape(B, Tq, NHEADS, hd).transpose(0, 2, 1, 3)
    k = k.reshape(B, Tk, NHEADS, hd).transpose(0, 2, 1, 3)
    v = v.reshape(B, Tk, NHEADS, hd).transpose(0, 2, 1, 3)
    attn = jax.nn.softmax((q @ k.transpose(0, 1, 3, 2)) * (hd ** -0.5), axis=-1)
    o = (attn @ v).transpose(0, 2, 1, 3).reshape(B, Tq, C)
    return o @ cp['out_w'] + cp['out_b']


def _hcl_body(imf_ref, pcf_ref, out_ref, acc_ref):
    b = pl.program_id(0)
    nb = pl.num_programs(0)
    imf = imf_ref[0]
    pcf = pcf_ref[0]
    imf = imf / jnp.maximum(jnp.sqrt(jnp.sum(imf * imf, -1, keepdims=True)), 1e-12)
    pcf = pcf / jnp.maximum(jnp.sqrt(jnp.sum(pcf * pcf, -1, keepdims=True)), 1e-12)
    T, C = imf.shape
    gc = jax.lax.dot_general(imf, pcf, (((0,), (0,)), ((), ())),
                             preferred_element_type=jnp.float32)
    eye = (jax.lax.broadcasted_iota(jnp.int32, (C, C), 0)
           == jax.lax.broadcasted_iota(jnp.int32, (C, C), 1)).astype(jnp.float32)
    gsum = jnp.sum((gc - eye) ** 2)
    il = jax.lax.dot_general(imf, imf, (((1,), (1,)), ((), ())),
                             preferred_element_type=jnp.float32)
    plm = jax.lax.dot_general(pcf, pcf, (((1,), (1,)), ((), ())),
                              preferred_element_type=jnp.float32)
    lsum = jnp.sum((il - plm) ** 2)

    @pl.when(b == 0)
    def _():
        acc_ref[0, 0] = 0.0
        acc_ref[0, 1] = 0.0

    acc_ref[0, 0] += gsum
    acc_ref[0, 1] += lsum

    @pl.when(b == nb - 1)
    def _():
        total = (acc_ref[0, 0] / (nb * C * C)
                 + acc_ref[0, 1] / (nb * T * T))
        out_ref[...] = jnp.full((1, 1), total, jnp.float32)


def _hcl_pallas(imf, pcf):
    B, T, C = imf.shape
    out = pl.pallas_call(
        _hcl_body,
        grid=(B,),
        in_specs=[
            pl.BlockSpec((1, T, C), lambda b: (b, 0, 0)),
            pl.BlockSpec((1, T, C), lambda b: (b, 0, 0)),
        ],
        out_specs=pl.BlockSpec((1, 1), lambda b: (0, 0)),
        out_shape=jax.ShapeDtypeStruct((1, 1), jnp.float32),
        scratch_shapes=[pltpu.SMEM((1, 2), jnp.float32)],
    )(imf, pcf)
    return out[0, 0]


def _forward(pc, im, params):
    p, f = pc, pc
    for sp in params['pc_stages']:
        p, f = _pc_stage(p, f, sp)
    pc_tok = f @ params['proj_w'] + params['proj_b']
    pos = _gelu(p @ params['pos1_w'] + params['pos1_b']) @ params['pos2_w'] + params['pos2_b']
    pc_tok = pc_tok + pos
    B, C, H, W = im.shape
    gh, gw = H // PATCH, W // PATCH
    x = im.reshape(B, C, gh, PATCH, gw, PATCH).transpose(0, 2, 4, 1, 3, 5).reshape(B, gh * gw, C * PATCH * PATCH)
    im_tok = x @ params['patch_w'] + params['patch_b']
    for bp in params['blocks']:
        im_tok = _vit_block(im_tok, bp)
    for bp in params['blocks']:
        pc_tok = _vit_block(pc_tok, bp)
    im_f = _ln(im_tok, params['im_norm_g'], params['im_norm_b'], 1e-6)
    pc_f = _ln(pc_tok, params['pc_norm_g'], params['pc_norm_b'], 1e-6)
    cross = _cross_mha(im_f, pc_f, params['cross'])
    cross = _ln(im_f + cross, params['cn_g'], params['cn_b'], 1e-5)
    fused = jnp.concatenate([im_f, cross], axis=-1) @ params['fuse_w'] + params['fuse_b']
    return _hcl_pallas(fused, pc_f)


def kernel(pc, im, params):
    return _forward(pc, im, params)


# speedup vs baseline: 1.7004x; 1.6983x over previous
import functools

import jax
import jax.numpy as jnp
from jax.experimental import pallas as pl
from jax.experimental.pallas import tpu as pltpu

D = 192
NHEADS = 12
DEPTH = 3
PATCH = 14
SCALE = 2
GROUP_SIZE = 32
RADIUS = 0.1


def _ln(x, g, b, eps):
    m = x.mean(-1, keepdims=True)
    v = x.var(-1, keepdims=True)
    return (x - m) / jnp.sqrt(v + eps) * g + b


def _inorm(x):
    m = x.mean(axis=(1, 2), keepdims=True)
    v = x.var(axis=(1, 2), keepdims=True)
    return (x - m) / jnp.sqrt(v + 1e-5)


def _gelu(x):
    return jax.nn.gelu(x, approximate=False)


def _fps_body(px_ref, py_ref, pz_ref, out_ref, *, n_samples):
    x = px_ref[...]
    y = py_ref[...]
    z = pz_ref[...]
    B, N = x.shape
    M = n_samples
    lane = jax.lax.broadcasted_iota(jnp.int32, (B, N), 1)
    col_m = jax.lax.broadcasted_iota(jnp.int32, (B, M), 1)
    lx = x[:, 0:1]
    ly = y[:, 0:1]
    lz = z[:, 0:1]
    dists0 = jnp.full((B, N), 1e10, jnp.float32)
    buf0 = jnp.zeros((B, M), jnp.int32)

    def body(i, carry):
        dists, lx, ly, lz, buf = carry
        dx = x - lx
        dy = y - ly
        dz = z - lz
        d = dx * dx + dy * dy + dz * dz
        dists = jnp.minimum(dists, d)
        maxv = jnp.max(dists, axis=1, keepdims=True)
        cand = jnp.where(dists == maxv, lane, N)
        nxt = jnp.min(cand, axis=1, keepdims=True)
        onehot = lane == nxt
        zero = jnp.zeros((), jnp.float32)
        lx = jnp.sum(jnp.where(onehot, x, zero), axis=1, keepdims=True)
        ly = jnp.sum(jnp.where(onehot, y, zero), axis=1, keepdims=True)
        lz = jnp.sum(jnp.where(onehot, z, zero), axis=1, keepdims=True)
        buf = jnp.where(col_m == i, nxt, buf)
        return (dists, lx, ly, lz, buf)

    carry = jax.lax.fori_loop(1, M, body, (dists0, lx, ly, lz, buf0))
    out_ref[...] = carry[4]


def _fps(p, n_samples):
    p = jax.lax.stop_gradient(p)
    B, N, _ = p.shape
    pt = p.transpose(0, 2, 1)
    px, py, pz = pt[:, 0], pt[:, 1], pt[:, 2]
    return pl.pallas_call(
        functools.partial(_fps_body, n_samples=n_samples),
        out_shape=jax.ShapeDtypeStruct((B, n_samples), jnp.int32),
    )(px, py, pz)


def _ball_query(support, centers, radius, nsample):
    support = jax.lax.stop_gradient(support)
    centers = jax.lax.stop_gradient(centers)
    B, N, _ = support.shape
    d = (jnp.sum(centers ** 2, -1)[:, :, None]
         + jnp.sum(support ** 2, -1)[:, None, :]
         - 2.0 * jnp.einsum('bmd,bnd->bmn', centers, support))
    idx = jnp.broadcast_to(jnp.arange(N, dtype=jnp.int32), d.shape)
    idx = jnp.where(d > radius * radius, N, idx)
    idx = jnp.sort(idx, axis=-1)[:, :, :nsample]
    first = idx[:, :, :1]
    idx = jnp.where(idx == N, jnp.broadcast_to(first, idx.shape), idx)
    return idx


def _gather(x, idx):
    return jnp.take_along_axis(x, idx[..., None], axis=1)


def _group(x, idx):
    B, M, K = idx.shape
    g = jnp.take_along_axis(x, idx.reshape(B, M * K)[..., None], axis=1)
    return g.reshape(B, M, K, x.shape[-1])


def _conv_block(x, w):
    return jax.nn.relu(_inorm(x @ w))


def _channel_attn(x, w1, w2):
    s = x.mean(axis=(1, 2))
    a = jax.nn.sigmoid(jax.nn.relu(s @ w1) @ w2)
    return x * a[:, None, None, :]


def _pc_stage(cur_p, cur_f, sp):
    B, N, _ = cur_p.shape
    M = N // SCALE
    idx = _fps(cur_p, M)
    center_p = _gather(cur_p, idx)
    center_f = _gather(cur_f, idx)
    nidx = _ball_query(cur_p, center_p, RADIUS, GROUP_SIZE)
    dp = _group(cur_p, nidx) - center_p[:, :, None, :]
    fj = _group(cur_f, nidx)
    df = fj - center_f[:, :, None, :]
    h = jnp.concatenate([dp, df], axis=-1)
    h = _channel_attn(_conv_block(h, sp['w1']), sp['ca1_w1'], sp['ca1_w2'])
    pooled = jnp.broadcast_to(h.max(axis=2, keepdims=True), h.shape)
    h = jnp.concatenate([pooled, h], axis=-1)
    h = _channel_attn(_conv_block(h, sp['w2']), sp['ca2_w1'], sp['ca2_w2'])
    return center_p, h.max(axis=2)


def _vit_block(x, bp):
    B, T, C = x.shape
    hd = C // NHEADS
    h = _ln(x, bp['ln1_g'], bp['ln1_b'], 1e-5)
    qkv = h @ bp['qkv_w'] + bp['qkv_b']
    q, k, v = jnp.split(qkv, 3, axis=-1)
    q = q.reshape(B, T, NHEADS, hd).transpose(0, 2, 1, 3)
    k = k.reshape(B, T, NHEADS, hd).transpose(0, 2, 1, 3)
    v = v.reshape(B, T, NHEADS, hd).transpose(0, 2, 1, 3)
    attn = jax.nn.softmax((q @ k.transpose(0, 1, 3, 2)) * (hd ** -0.5), axis=-1)
    o = (attn @ v).transpose(0, 2, 1, 3).reshape(B, T, C)
    x = x + (o @ bp['proj_w'] + bp['proj_b'])
    h = _ln(x, bp['ln2_g'], bp['ln2_b'], 1e-5)
    x = x + (_gelu(h @ bp['fc1_w'] + bp['fc1_b']) @ bp['fc2_w'] + bp['fc2_b'])
    return x


def _cross_mha(q_in, kv_in, cp):
    B, Tq, C = q_in.shape
    Tk = kv_in.shape[1]
    hd = C // NHEADS
    q = q_in @ cp['in_w'][:, :C] + cp['in_b'][:C]
    k = kv_in @ cp['in_w'][:, C:2 * C] + cp['in_b'][C:2 * C]
    v = kv_in @ cp['in_w'][:, 2 * C:] + cp['in_b'][2 * C:]
    q = q.reshape(B, Tq, NHEADS, hd).transpose(0, 2, 1, 3)
    k = k.reshape(B, Tk, NHEADS, hd).transpose(0, 2, 1, 3)
    v = v.reshape(B, Tk, NHEADS, hd).transpose(0, 2, 1, 3)
    attn = jax.nn.softmax((q @ k.transpose(0, 1, 3, 2)) * (hd ** -0.5), axis=-1)
    o = (attn @ v).transpose(0, 2, 1, 3).reshape(B, Tq, C)
    return o @ cp['out_w'] + cp['out_b']


def _hcl_body(imf_ref, pcf_ref, out_ref, acc_ref):
    b = pl.program_id(0)
    nb = pl.num_programs(0)
    imf = imf_ref[0]
    pcf = pcf_ref[0]
    imf = imf / jnp.maximum(jnp.sqrt(jnp.sum(imf * imf, -1, keepdims=True)), 1e-12)
    pcf = pcf / jnp.maximum(jnp.sqrt(jnp.sum(pcf * pcf, -1, keepdims=True)), 1e-12)
    T, C = imf.shape
    gc = jax.lax.dot_general(imf, pcf, (((0,), (0,)), ((), ())),
                             preferred_element_type=jnp.float32)
    eye = (jax.lax.broadcasted_iota(jnp.int32, (C, C), 0)
           == jax.lax.broadcasted_iota(jnp.int32, (C, C), 1)).astype(jnp.float32)
    gsum = jnp.sum((gc - eye) ** 2)
    il = jax.lax.dot_general(imf, imf, (((1,), (1,)), ((), ())),
                             preferred_element_type=jnp.float32)
    plm = jax.lax.dot_general(pcf, pcf, (((1,), (1,)), ((), ())),
                              preferred_element_type=jnp.float32)
    lsum = jnp.sum((il - plm) ** 2)

    @pl.when(b == 0)
    def _():
        acc_ref[0, 0] = 0.0
        acc_ref[0, 1] = 0.0

    acc_ref[0, 0] += gsum
    acc_ref[0, 1] += lsum

    @pl.when(b == nb - 1)
    def _():
        total = (acc_ref[0, 0] / (nb * C * C)
                 + acc_ref[0, 1] / (nb * T * T))
        out_ref[...] = jnp.full((1, 1), total, jnp.float32)


def _hcl_pallas(imf, pcf):
    B, T, C = imf.shape
    out = pl.pallas_call(
        _hcl_body,
        grid=(B,),
        in_specs=[
            pl.BlockSpec((1, T, C), lambda b: (b, 0, 0)),
            pl.BlockSpec((1, T, C), lambda b: (b, 0, 0)),
        ],
        out_specs=pl.BlockSpec((1, 1), lambda b: (0, 0)),
        out_shape=jax.ShapeDtypeStruct((1, 1), jnp.float32),
        scratch_shapes=[pltpu.SMEM((1, 2), jnp.float32)],
    )(imf, pcf)
    return out[0, 0]


def _forward(pc, im, params):
    p, f = pc, pc
    for sp in params['pc_stages']:
        p, f = _pc_stage(p, f, sp)
    pc_tok = f @ params['proj_w'] + params['proj_b']
    pos = _gelu(p @ params['pos1_w'] + params['pos1_b']) @ params['pos2_w'] + params['pos2_b']
    pc_tok = pc_tok + pos
    B, C, H, W = im.shape
    gh, gw = H // PATCH, W // PATCH
    x = im.reshape(B, C, gh, PATCH, gw, PATCH).transpose(0, 2, 4, 1, 3, 5).reshape(B, gh * gw, C * PATCH * PATCH)
    im_tok = x @ params['patch_w'] + params['patch_b']
    for bp in params['blocks']:
        im_tok = _vit_block(im_tok, bp)
    for bp in params['blocks']:
        pc_tok = _vit_block(pc_tok, bp)
    im_f = _ln(im_tok, params['im_norm_g'], params['im_norm_b'], 1e-6)
    pc_f = _ln(pc_tok, params['pc_norm_g'], params['pc_norm_b'], 1e-6)
    cross = _cross_mha(im_f, pc_f, params['cross'])
    cross = _ln(im_f + cross, params['cn_g'], params['cn_b'], 1e-5)
    fused = jnp.concatenate([im_f, cross], axis=-1) @ params['fuse_w'] + params['fuse_b']
    return _hcl_pallas(fused, pc_f)


def kernel(pc, im, params):
    return _forward(pc, im, params)


# speedup vs baseline: 1.9269x; 1.1332x over previous
import functools

import jax
import jax.numpy as jnp
from jax import lax
from jax.experimental import pallas as pl
from jax.experimental.pallas import tpu as pltpu
from jax.experimental.pallas import tpu_sc as plsc

D = 192
NHEADS = 12
DEPTH = 3
PATCH = 14
SCALE = 2
GROUP_SIZE = 32
RADIUS = 0.1


def _ln(x, g, b, eps):
    m = x.mean(-1, keepdims=True)
    v = x.var(-1, keepdims=True)
    return (x - m) / jnp.sqrt(v + eps) * g + b


def _inorm(x):
    m = x.mean(axis=(1, 2), keepdims=True)
    v = x.var(axis=(1, 2), keepdims=True)
    return (x - m) / jnp.sqrt(v + 1e-5)


def _gelu(x):
    return jax.nn.gelu(x, approximate=False)


def _fps_body(px_ref, py_ref, pz_ref, oi_ref, ox_ref, oy_ref, oz_ref, *,
              n_samples):
    x = px_ref[...]
    y = py_ref[...]
    z = pz_ref[...]
    B, N = x.shape
    M = n_samples
    lane = jax.lax.broadcasted_iota(jnp.int32, (B, N), 1)
    col_m = jax.lax.broadcasted_iota(jnp.int32, (B, M), 1)
    lx = x[:, 0:1]
    ly = y[:, 0:1]
    lz = z[:, 0:1]
    dists0 = jnp.full((B, N), 1e10, jnp.float32)
    buf0 = jnp.zeros((B, M), jnp.int32)
    col0 = col_m == 0
    bx0 = jnp.where(col0, lx, 0.0)
    by0 = jnp.where(col0, ly, 0.0)
    bz0 = jnp.where(col0, lz, 0.0)

    def body(i, carry):
        dists, lx, ly, lz, buf, bx, by, bz = carry
        dx = x - lx
        dy = y - ly
        dz = z - lz
        d = dx * dx + dy * dy + dz * dz
        dists = jnp.minimum(dists, d)
        maxv = jnp.max(dists, axis=1, keepdims=True)
        cand = jnp.where(dists == maxv, lane, N)
        nxt = jnp.min(cand, axis=1, keepdims=True)
        onehot = lane == nxt
        zero = jnp.zeros((), jnp.float32)
        lx = jnp.sum(jnp.where(onehot, x, zero), axis=1, keepdims=True)
        ly = jnp.sum(jnp.where(onehot, y, zero), axis=1, keepdims=True)
        lz = jnp.sum(jnp.where(onehot, z, zero), axis=1, keepdims=True)
        coli = col_m == i
        buf = jnp.where(coli, nxt, buf)
        bx = jnp.where(coli, lx, bx)
        by = jnp.where(coli, ly, by)
        bz = jnp.where(coli, lz, bz)
        return (dists, lx, ly, lz, buf, bx, by, bz)

    carry = jax.lax.fori_loop(
        1, M, body, (dists0, lx, ly, lz, buf0, bx0, by0, bz0))
    oi_ref[...] = carry[4]
    ox_ref[...] = carry[5]
    oy_ref[...] = carry[6]
    oz_ref[...] = carry[7]


def _fps(p, n_samples):
    p = jax.lax.stop_gradient(p)
    B, N, _ = p.shape
    pt = p.transpose(0, 2, 1)
    px, py, pz = pt[:, 0], pt[:, 1], pt[:, 2]
    idx, cx, cy, cz = pl.pallas_call(
        functools.partial(_fps_body, n_samples=n_samples),
        out_shape=(
            jax.ShapeDtypeStruct((B, n_samples), jnp.int32),
            jax.ShapeDtypeStruct((B, n_samples), jnp.float32),
            jax.ShapeDtypeStruct((B, n_samples), jnp.float32),
            jax.ShapeDtypeStruct((B, n_samples), jnp.float32),
        ),
    )(px, py, pz)
    return idx, jnp.stack([cx, cy, cz], axis=-1)


def _bq_body(cen_ref, sup_ref, out_ref, *, radius, nsample, n):
    cen = cen_ref[0]
    sup = sup_ref[0]
    dot = jax.lax.dot_general(cen, sup, (((1,), (0,)), ((), ())),
                              preferred_element_type=jnp.float32)
    c2 = jnp.sum(cen * cen, axis=1, keepdims=True)
    s2 = jnp.sum(sup * sup, axis=0, keepdims=True)
    d = (c2 + s2) - 2.0 * dot
    mblk = d.shape[0]
    lane = jax.lax.broadcasted_iota(jnp.int32, (mblk, n), 1)
    idx0 = jnp.where(d > radius * radius, n, lane)
    kcol = jax.lax.broadcasted_iota(jnp.int32, (mblk, nsample), 1)
    acc0 = jnp.zeros((mblk, nsample), jnp.int32)

    def body(k, carry):
        idx, acc = carry
        m = jnp.min(idx, axis=1, keepdims=True)
        acc = jnp.where(kcol == k, m, acc)
        idx = jnp.where(idx == m, n, idx)
        return (idx, acc)

    _, acc = jax.lax.fori_loop(0, nsample, body, (idx0, acc0))
    first = acc[:, 0:1]
    out_ref[0] = jnp.where(acc == n, first, acc)


def _ball_query(support, centers, radius, nsample):
    support = jax.lax.stop_gradient(support)
    centers = jax.lax.stop_gradient(centers)
    B, N, _ = support.shape
    M = centers.shape[1]
    sup_t = support.transpose(0, 2, 1)
    mblk = min(M, 256)
    return pl.pallas_call(
        functools.partial(_bq_body, radius=radius, nsample=nsample, n=N),
        grid=(B, M // mblk),
        in_specs=[
            pl.BlockSpec((1, mblk, 3), lambda b, mb: (b, mb, 0)),
            pl.BlockSpec((1, 3, N), lambda b, mb: (b, 0, 0)),
        ],
        out_specs=pl.BlockSpec((1, mblk, nsample), lambda b, mb: (b, mb, 0)),
        out_shape=jax.ShapeDtypeStruct((B, M, nsample), jnp.int32),
    )(centers, sup_t)


def _gather(x, idx):
    return jnp.take_along_axis(x, idx[..., None], axis=1)


def _group(x, idx):
    B, M, K = idx.shape
    g = jnp.take_along_axis(x, idx.reshape(B, M * K)[..., None], axis=1)
    return g.reshape(B, M, K, x.shape[-1])


def _conv_block(x, w):
    return jax.nn.relu(_inorm(x @ w))


def _channel_attn(x, w1, w2):
    s = x.mean(axis=(1, 2))
    a = jax.nn.sigmoid(jax.nn.relu(s @ w1) @ w2)
    return x * a[:, None, None, :]


def _pc_stage(cur_p, cur_f, sp):
    B, N, _ = cur_p.shape
    M = N // SCALE
    idx, center_p = _fps(cur_p, M)
    center_f = center_p if cur_f is cur_p else _gather(cur_f, idx)
    nidx = _ball_query(cur_p, center_p, RADIUS, GROUP_SIZE)
    dp = _group(cur_p, nidx) - center_p[:, :, None, :]
    fj = _group(cur_f, nidx)
    df = fj - center_f[:, :, None, :]
    h = jnp.concatenate([dp, df], axis=-1)
    h = _channel_attn(_conv_block(h, sp['w1']), sp['ca1_w1'], sp['ca1_w2'])
    pooled = jnp.broadcast_to(h.max(axis=2, keepdims=True), h.shape)
    h = jnp.concatenate([pooled, h], axis=-1)
    h = _channel_attn(_conv_block(h, sp['w2']), sp['ca2_w1'], sp['ca2_w2'])
    return center_p, h.max(axis=2)


def _vit_block(x, bp):
    B, T, C = x.shape
    hd = C // NHEADS
    h = _ln(x, bp['ln1_g'], bp['ln1_b'], 1e-5)
    qkv = h @ bp['qkv_w'] + bp['qkv_b']
    q, k, v = jnp.split(qkv, 3, axis=-1)
    q = q.reshape(B, T, NHEADS, hd).transpose(0, 2, 1, 3)
    k = k.reshape(B, T, NHEADS, hd).transpose(0, 2, 1, 3)
    v = v.reshape(B, T, NHEADS, hd).transpose(0, 2, 1, 3)
    attn = jax.nn.softmax((q @ k.transpose(0, 1, 3, 2)) * (hd ** -0.5), axis=-1)
    o = (attn @ v).transpose(0, 2, 1, 3).reshape(B, T, C)
    x = x + (o @ bp['proj_w'] + bp['proj_b'])
    h = _ln(x, bp['ln2_g'], bp['ln2_b'], 1e-5)
    x = x + (_gelu(h @ bp['fc1_w'] + bp['fc1_b']) @ bp['fc2_w'] + bp['fc2_b'])
    return x


def _cross_mha(q_in, kv_in, cp):
    B, Tq, C = q_in.shape
    Tk = kv_in.shape[1]
    hd = C // NHEADS
    q = q_in @ cp['in_w'][:, :C] + cp['in_b'][:C]
    k = kv_in @ cp['in_w'][:, C:2 * C] + cp['in_b'][C:2 * C]
    v = kv_in @ cp['in_w'][:, 2 * C:] + cp['in_b'][2 * C:]
    q = q.reshape(B, Tq, NHEADS, hd).transpose(0, 2, 1, 3)
    k = k.reshape(B, Tk, NHEADS, hd).transpose(0, 2, 1, 3)
    v = v.reshape(B, Tk, NHEADS, hd).transpose(0, 2, 1, 3)
    attn = jax.nn.softmax((q @ k.transpose(0, 1, 3, 2)) * (hd ** -0.5), axis=-1)
    o = (attn @ v).transpose(0, 2, 1, 3).reshape(B, Tq, C)
    return o @ cp['out_w'] + cp['out_b']


def _hcl_body(imf_ref, pcf_ref, out_ref, acc_ref):
    b = pl.program_id(0)
    nb = pl.num_programs(0)
    imf = imf_ref[0]
    pcf = pcf_ref[0]
    imf = imf / jnp.maximum(jnp.sqrt(jnp.sum(imf * imf, -1, keepdims=True)), 1e-12)
    pcf = pcf / jnp.maximum(jnp.sqrt(jnp.sum(pcf * pcf, -1, keepdims=True)), 1e-12)
    T, C = imf.shape
    gc = jax.lax.dot_general(imf, pcf, (((0,), (0,)), ((), ())),
                             preferred_element_type=jnp.float32)
    eye = (jax.lax.broadcasted_iota(jnp.int32, (C, C), 0)
           == jax.lax.broadcasted_iota(jnp.int32, (C, C), 1)).astype(jnp.float32)
    gsum = jnp.sum((gc - eye) ** 2)
    il = jax.lax.dot_general(imf, imf, (((1,), (1,)), ((), ())),
                             preferred_element_type=jnp.float32)
    plm = jax.lax.dot_general(pcf, pcf, (((1,), (1,)), ((), ())),
                              preferred_element_type=jnp.float32)
    lsum = jnp.sum((il - plm) ** 2)

    @pl.when(b == 0)
    def _():
        acc_ref[0, 0] = 0.0
        acc_ref[0, 1] = 0.0

    acc_ref[0, 0] += gsum
    acc_ref[0, 1] += lsum

    @pl.when(b == nb - 1)
    def _():
        total = (acc_ref[0, 0] / (nb * C * C)
                 + acc_ref[0, 1] / (nb * T * T))
        out_ref[...] = jnp.full((1, 1), total, jnp.float32)


def _hcl_pallas(imf, pcf):
    B, T, C = imf.shape
    out = pl.pallas_call(
        _hcl_body,
        grid=(B,),
        in_specs=[
            pl.BlockSpec((1, T, C), lambda b: (b, 0, 0)),
            pl.BlockSpec((1, T, C), lambda b: (b, 0, 0)),
        ],
        out_specs=pl.BlockSpec((1, 1), lambda b: (0, 0)),
        out_shape=jax.ShapeDtypeStruct((1, 1), jnp.float32),
        scratch_shapes=[pltpu.SMEM((1, 2), jnp.float32)],
    )(imf, pcf)
    return out[0, 0]


def _forward(pc, im, params):
    p, f = pc, pc
    for sp in params['pc_stages']:
        p, f = _pc_stage(p, f, sp)
    pc_tok = f @ params['proj_w'] + params['proj_b']
    pos = _gelu(p @ params['pos1_w'] + params['pos1_b']) @ params['pos2_w'] + params['pos2_b']
    pc_tok = pc_tok + pos
    B, C, H, W = im.shape
    gh, gw = H // PATCH, W // PATCH
    x = im.reshape(B, C, gh, PATCH, gw, PATCH).transpose(0, 2, 4, 1, 3, 5).reshape(B, gh * gw, C * PATCH * PATCH)
    im_tok = x @ params['patch_w'] + params['patch_b']
    for bp in params['blocks']:
        im_tok = _vit_block(im_tok, bp)
    for bp in params['blocks']:
        pc_tok = _vit_block(pc_tok, bp)
    im_f = _ln(im_tok, params['im_norm_g'], params['im_norm_b'], 1e-6)
    pc_f = _ln(pc_tok, params['pc_norm_g'], params['pc_norm_b'], 1e-6)
    cross = _cross_mha(im_f, pc_f, params['cross'])
    cross = _ln(im_f + cross, params['cn_g'], params['cn_b'], 1e-5)
    fused = jnp.concatenate([im_f, cross], axis=-1) @ params['fuse_w'] + params['fuse_b']
    return _hcl_pallas(fused, pc_f)


def kernel(pc, im, params):
    return _forward(pc, im, params)


# speedup vs baseline: 5.1078x; 2.6508x over previous
import functools

import jax
import jax.numpy as jnp
from jax import lax
from jax.experimental import pallas as pl
from jax.experimental.pallas import tpu as pltpu
from jax.experimental.pallas import tpu_sc as plsc

D = 192
NHEADS = 12
DEPTH = 3
PATCH = 14
SCALE = 2
GROUP_SIZE = 32
RADIUS = 0.1


def _ln(x, g, b, eps):
    m = x.mean(-1, keepdims=True)
    v = x.var(-1, keepdims=True)
    return (x - m) / jnp.sqrt(v + eps) * g + b


def _inorm(x):
    m = x.mean(axis=(1, 2), keepdims=True)
    v = x.var(axis=(1, 2), keepdims=True)
    return (x - m) / jnp.sqrt(v + 1e-5)


def _gelu(x):
    return jax.nn.gelu(x, approximate=False)


def _fps_body(px_ref, py_ref, pz_ref, oi_ref, ox_ref, oy_ref, oz_ref, *,
              n_samples):
    x = px_ref[...]
    y = py_ref[...]
    z = pz_ref[...]
    B, N = x.shape
    M = n_samples
    lane = jax.lax.broadcasted_iota(jnp.int32, (B, N), 1)
    col_m = jax.lax.broadcasted_iota(jnp.int32, (B, M), 1)
    lx = x[:, 0:1]
    ly = y[:, 0:1]
    lz = z[:, 0:1]
    dists0 = jnp.full((B, N), 1e10, jnp.float32)
    buf0 = jnp.zeros((B, M), jnp.int32)
    col0 = col_m == 0
    bx0 = jnp.where(col0, lx, 0.0)
    by0 = jnp.where(col0, ly, 0.0)
    bz0 = jnp.where(col0, lz, 0.0)

    def body(i, carry):
        dists, lx, ly, lz, buf, bx, by, bz = carry
        dx = x - lx
        dy = y - ly
        dz = z - lz
        d = dx * dx + dy * dy + dz * dz
        dists = jnp.minimum(dists, d)
        maxv = jnp.max(dists, axis=1, keepdims=True)
        cand = jnp.where(dists == maxv, lane, N)
        nxt = jnp.min(cand, axis=1, keepdims=True)
        onehot = lane == nxt
        zero = jnp.zeros((), jnp.float32)
        lx = jnp.sum(jnp.where(onehot, x, zero), axis=1, keepdims=True)
        ly = jnp.sum(jnp.where(onehot, y, zero), axis=1, keepdims=True)
        lz = jnp.sum(jnp.where(onehot, z, zero), axis=1, keepdims=True)
        coli = col_m == i
        buf = jnp.where(coli, nxt, buf)
        bx = jnp.where(coli, lx, bx)
        by = jnp.where(coli, ly, by)
        bz = jnp.where(coli, lz, bz)
        return (dists, lx, ly, lz, buf, bx, by, bz)

    carry = jax.lax.fori_loop(
        1, M, body, (dists0, lx, ly, lz, buf0, bx0, by0, bz0))
    oi_ref[...] = carry[4]
    ox_ref[...] = carry[5]
    oy_ref[...] = carry[6]
    oz_ref[...] = carry[7]


def _fps(p, n_samples):
    p = jax.lax.stop_gradient(p)
    B, N, _ = p.shape
    pt = p.transpose(0, 2, 1)
    px, py, pz = pt[:, 0], pt[:, 1], pt[:, 2]
    idx, cx, cy, cz = pl.pallas_call(
        functools.partial(_fps_body, n_samples=n_samples),
        out_shape=(
            jax.ShapeDtypeStruct((B, n_samples), jnp.int32),
            jax.ShapeDtypeStruct((B, n_samples), jnp.float32),
            jax.ShapeDtypeStruct((B, n_samples), jnp.float32),
            jax.ShapeDtypeStruct((B, n_samples), jnp.float32),
        ),
    )(px, py, pz)
    return idx, jnp.stack([cx, cy, cz], axis=-1)


def _bq_body(cen_ref, sup_ref, out_ref, *, radius, nsample, n):
    cen = cen_ref[0]
    sup = sup_ref[0]
    dot = jax.lax.dot_general(cen, sup, (((1,), (0,)), ((), ())),
                              preferred_element_type=jnp.float32)
    c2 = jnp.sum(cen * cen, axis=1, keepdims=True)
    s2 = jnp.sum(sup * sup, axis=0, keepdims=True)
    d = (c2 + s2) - 2.0 * dot
    mblk = d.shape[0]
    lane = jax.lax.broadcasted_iota(jnp.int32, (mblk, n), 1)
    idx0 = jnp.where(d > radius * radius, n, lane)
    kcol = jax.lax.broadcasted_iota(jnp.int32, (mblk, nsample), 1)
    acc0 = jnp.zeros((mblk, nsample), jnp.int32)

    def body(k, carry):
        idx, acc = carry
        m = jnp.min(idx, axis=1, keepdims=True)
        acc = jnp.where(kcol == k, m, acc)
        idx = jnp.where(idx == m, n, idx)
        return (idx, acc)

    _, acc = jax.lax.fori_loop(0, nsample, body, (idx0, acc0))
    first = acc[:, 0:1]
    out_ref[0] = jnp.where(acc == n, first, acc)


def _ball_query(support, centers, radius, nsample):
    support = jax.lax.stop_gradient(support)
    centers = jax.lax.stop_gradient(centers)
    B, N, _ = support.shape
    M = centers.shape[1]
    sup_t = support.transpose(0, 2, 1)
    mblk = min(M, 256)
    return pl.pallas_call(
        functools.partial(_bq_body, radius=radius, nsample=nsample, n=N),
        grid=(B, M // mblk),
        in_specs=[
            pl.BlockSpec((1, mblk, 3), lambda b, mb: (b, mb, 0)),
            pl.BlockSpec((1, 3, N), lambda b, mb: (b, 0, 0)),
        ],
        out_specs=pl.BlockSpec((1, mblk, nsample), lambda b, mb: (b, mb, 0)),
        out_shape=jax.ShapeDtypeStruct((B, M, nsample), jnp.int32),
    )(centers, sup_t)


_SC_INFO = plsc.get_sparse_core_info()
_NW = _SC_INFO.num_cores * _SC_INFO.num_subcores
_SC_G = 512
_SC_UNIT = _NW * _SC_G


def _sc_gather_body(table_hbm, idx_hbm, out_hbm, idx_v, rows_v, sem, *, rpw, g):
    wid = lax.axis_index("s") * _SC_INFO.num_cores + lax.axis_index("c")
    base = wid * rpw

    def outer(i, carry):
        off = base + i * g
        pltpu.sync_copy(idx_hbm.at[pl.ds(off, g)], idx_v)
        waits = []
        for j in range(g // 128):
            waits.append(pltpu.async_copy(
                table_hbm.at[idx_v.at[pl.ds(j * 128, 128)]],
                rows_v.at[pl.ds(j * 128, 128)], sem))
        for w in waits:
            w.wait()
        pltpu.sync_copy(rows_v, out_hbm.at[pl.ds(off, g)])
        return carry

    lax.fori_loop(0, rpw // g, outer, 0)


def _sc_gather(table, idx):
    rows, d = idx.shape[0], table.shape[1]
    kern = pl.kernel(
        functools.partial(_sc_gather_body, rpw=rows // _NW, g=_SC_G),
        out_type=jax.ShapeDtypeStruct((rows, d), jnp.float32),
        mesh=plsc.VectorSubcoreMesh(core_axis_name="c", subcore_axis_name="s"),
        scratch_types=[
            pltpu.VMEM((_SC_G,), jnp.int32),
            pltpu.VMEM((_SC_G, d), jnp.float32),
            pltpu.SemaphoreType.DMA,
        ],
        compiler_params=pltpu.CompilerParams(use_tc_tiling_on_sc=False),
    )
    return kern(table, idx)


def _pad_rows(ix):
    r = ix.shape[0]
    rp = -(-r // _SC_UNIT) * _SC_UNIT
    return jnp.pad(ix, (0, rp - r)) if rp != r else ix


def _pad16(p):
    return jnp.pad(p, ((0, 0), (0, 0), (0, 13)))


def _conv_block(x, w):
    return jax.nn.relu(_inorm(x @ w))


def _channel_attn(x, w1, w2):
    s = x.mean(axis=(1, 2))
    a = jax.nn.sigmoid(jax.nn.relu(s @ w1) @ w2)
    return x * a[:, None, None, :]


def _pc_stage(cur_p, cur_f, sp):
    B, N, _ = cur_p.shape
    M = N // SCALE
    K = GROUP_SIZE
    first = cur_f is cur_p
    idx, center_p = _fps(cur_p, M)
    nidx = _ball_query(cur_p, center_p, RADIUS, GROUP_SIZE)
    off = jnp.arange(B, dtype=jnp.int32)[:, None] * N
    gidx = (nidx.reshape(B, M * K) + off).reshape(-1)
    w1 = sp['w1']
    zpad = jnp.zeros((13, w1.shape[1]), jnp.float32)
    if first:
        table = _pad16(cur_p).reshape(B * N, 16)
        out = _sc_gather(table, _pad_rows(gidx))
        g = out[:B * M * K].reshape(B, M, K, 16)
        dp = g - _pad16(center_p)[:, :, None, :]
        h = jnp.concatenate([dp, dp], axis=-1)
        w1p = jnp.concatenate([w1[:3], zpad, w1[3:6], zpad], axis=0)
    else:
        c = cur_f.shape[-1]
        table = jnp.concatenate([_pad16(cur_p), cur_f], -1).reshape(B * N, 16 + c)
        cidx = (idx + off).reshape(-1)
        out = _sc_gather(table, _pad_rows(jnp.concatenate([gidx, cidx])))
        ng = B * M * K
        g = out[:ng].reshape(B, M, K, 16 + c)
        cen = out[ng:ng + B * M].reshape(B, M, 16 + c)
        h = g - cen[:, :, None, :]
        w1p = jnp.concatenate([w1[:3], zpad, w1[3:]], axis=0)
    h = _channel_attn(_conv_block(h, w1p), sp['ca1_w1'], sp['ca1_w2'])
    pooled = jnp.broadcast_to(h.max(axis=2, keepdims=True), h.shape)
    h = jnp.concatenate([pooled, h], axis=-1)
    h = _channel_attn(_conv_block(h, sp['w2']), sp['ca2_w1'], sp['ca2_w2'])
    return center_p, h.max(axis=2)


def _vit_block(x, bp):
    B, T, C = x.shape
    hd = C // NHEADS
    h = _ln(x, bp['ln1_g'], bp['ln1_b'], 1e-5)
    qkv = h @ bp['qkv_w'] + bp['qkv_b']
    q, k, v = jnp.split(qkv, 3, axis=-1)
    q = q.reshape(B, T, NHEADS, hd).transpose(0, 2, 1, 3)
    k = k.reshape(B, T, NHEADS, hd).transpose(0, 2, 1, 3)
    v = v.reshape(B, T, NHEADS, hd).transpose(0, 2, 1, 3)
    attn = jax.nn.softmax((q @ k.transpose(0, 1, 3, 2)) * (hd ** -0.5), axis=-1)
    o = (attn @ v).transpose(0, 2, 1, 3).reshape(B, T, C)
    x = x + (o @ bp['proj_w'] + bp['proj_b'])
    h = _ln(x, bp['ln2_g'], bp['ln2_b'], 1e-5)
    x = x + (_gelu(h @ bp['fc1_w'] + bp['fc1_b']) @ bp['fc2_w'] + bp['fc2_b'])
    return x


def _cross_mha(q_in, kv_in, cp):
    B, Tq, C = q_in.shape
    Tk = kv_in.shape[1]
    hd = C // NHEADS
    q = q_in @ cp['in_w'][:, :C] + cp['in_b'][:C]
    k = kv_in @ cp['in_w'][:, C:2 * C] + cp['in_b'][C:2 * C]
    v = kv_in @ cp['in_w'][:, 2 * C:] + cp['in_b'][2 * C:]
    q = q.reshape(B, Tq, NHEADS, hd).transpose(0, 2, 1, 3)
    k = k.reshape(B, Tk, NHEADS, hd).transpose(0, 2, 1, 3)
    v = v.reshape(B, Tk, NHEADS, hd).transpose(0, 2, 1, 3)
    attn = jax.nn.softmax((q @ k.transpose(0, 1, 3, 2)) * (hd ** -0.5), axis=-1)
    o = (attn @ v).transpose(0, 2, 1, 3).reshape(B, Tq, C)
    return o @ cp['out_w'] + cp['out_b']


def _hcl_body(imf_ref, pcf_ref, out_ref, acc_ref):
    b = pl.program_id(0)
    nb = pl.num_programs(0)
    imf = imf_ref[0]
    pcf = pcf_ref[0]
    imf = imf / jnp.maximum(jnp.sqrt(jnp.sum(imf * imf, -1, keepdims=True)), 1e-12)
    pcf = pcf / jnp.maximum(jnp.sqrt(jnp.sum(pcf * pcf, -1, keepdims=True)), 1e-12)
    T, C = imf.shape
    gc = jax.lax.dot_general(imf, pcf, (((0,), (0,)), ((), ())),
                             preferred_element_type=jnp.float32)
    eye = (jax.lax.broadcasted_iota(jnp.int32, (C, C), 0)
           == jax.lax.broadcasted_iota(jnp.int32, (C, C), 1)).astype(jnp.float32)
    gsum = jnp.sum((gc - eye) ** 2)
    il = jax.lax.dot_general(imf, imf, (((1,), (1,)), ((), ())),
                             preferred_element_type=jnp.float32)
    plm = jax.lax.dot_general(pcf, pcf, (((1,), (1,)), ((), ())),
                              preferred_element_type=jnp.float32)
    lsum = jnp.sum((il - plm) ** 2)

    @pl.when(b == 0)
    def _():
        acc_ref[0, 0] = 0.0
        acc_ref[0, 1] = 0.0

    acc_ref[0, 0] += gsum
    acc_ref[0, 1] += lsum

    @pl.when(b == nb - 1)
    def _():
        total = (acc_ref[0, 0] / (nb * C * C)
                 + acc_ref[0, 1] / (nb * T * T))
        out_ref[...] = jnp.full((1, 1), total, jnp.float32)


def _hcl_pallas(imf, pcf):
    B, T, C = imf.shape
    out = pl.pallas_call(
        _hcl_body,
        grid=(B,),
        in_specs=[
            pl.BlockSpec((1, T, C), lambda b: (b, 0, 0)),
            pl.BlockSpec((1, T, C), lambda b: (b, 0, 0)),
        ],
        out_specs=pl.BlockSpec((1, 1), lambda b: (0, 0)),
        out_shape=jax.ShapeDtypeStruct((1, 1), jnp.float32),
        scratch_shapes=[pltpu.SMEM((1, 2), jnp.float32)],
    )(imf, pcf)
    return out[0, 0]


def _forward(pc, im, params):
    p, f = pc, pc
    for sp in params['pc_stages']:
        p, f = _pc_stage(p, f, sp)
    pc_tok = f @ params['proj_w'] + params['proj_b']
    pos = _gelu(p @ params['pos1_w'] + params['pos1_b']) @ params['pos2_w'] + params['pos2_b']
    pc_tok = pc_tok + pos
    B, C, H, W = im.shape
    gh, gw = H // PATCH, W // PATCH
    x = im.reshape(B, C, gh, PATCH, gw, PATCH).transpose(0, 2, 4, 1, 3, 5).reshape(B, gh * gw, C * PATCH * PATCH)
    im_tok = x @ params['patch_w'] + params['patch_b']
    for bp in params['blocks']:
        im_tok = _vit_block(im_tok, bp)
    for bp in params['blocks']:
        pc_tok = _vit_block(pc_tok, bp)
    im_f = _ln(im_tok, params['im_norm_g'], params['im_norm_b'], 1e-6)
    pc_f = _ln(pc_tok, params['pc_norm_g'], params['pc_norm_b'], 1e-6)
    cross = _cross_mha(im_f, pc_f, params['cross'])
    cross = _ln(im_f + cross, params['cn_g'], params['cn_b'], 1e-5)
    fused = jnp.concatenate([im_f, cross], axis=-1) @ params['fuse_w'] + params['fuse_b']
    return _hcl_pallas(fused, pc_f)


def kernel(pc, im, params):
    return _forward(pc, im, params)
